# baseline (speedup 1.0000x reference)
import jax
import jax.numpy as jnp
from jax.experimental import pallas as pl
from jax.experimental.pallas import tpu as pltpu

_NSLAB = 8


def _gnn_kernel(adj_hbm, x_ref, W1_ref, b1_ref, W2_ref, b2_ref,
                out_ref, slabs, ah, sems):
    B = adj_hbm.shape[0]
    n = adj_hbm.shape[1]
    rows = n // _NSLAB

    def start(g, s):
        for i in range(_NSLAB):
            pltpu.make_async_copy(
                adj_hbm.at[g, pl.ds(i * rows, rows), :],
                slabs.at[s, i], sems.at[s, i]).start()

    r_i = jax.lax.broadcasted_iota(jnp.int32, (rows, rows), 0)
    c_i = jax.lax.broadcasted_iota(jnp.int32, (rows, rows), 1)
    eye_small = (r_i == c_i).astype(jnp.float32)

    def land(g, s):
        chunks = []
        for i in range(_NSLAB):
            pltpu.make_async_copy(
                adj_hbm.at[g, pl.ds(i * rows, rows), :],
                slabs.at[s, i], sems.at[s, i]).wait()
            slab = slabs[s, i]
            ah[pl.ds(i * rows, rows), :] = slab.astype(jnp.bfloat16)
            dblk = slab[:, i * rows:(i + 1) * rows] * eye_small
            chunks.append(jnp.sum(dblk, axis=0, keepdims=True))
        return jnp.concatenate(chunks, axis=1)

    start(0, 0)
    if B > 1:
        start(1, 1)

    W1T = jnp.transpose(W1_ref[...]).astype(jnp.bfloat16)
    W2T = jnp.transpose(W2_ref[...]).astype(jnp.bfloat16)
    b1c = b1_ref[...].reshape(-1, 1)
    b2c = b2_ref[...].reshape(-1, 1)
    ones8 = jnp.full((8, n), 1.0, dtype=jnp.bfloat16)

    for g in range(B):
        diag = land(g, g % 2)
        if g + 2 < B:
            start(g + 2, g % 2)

        adj_bf = ah[...]
        colsum = jnp.dot(ones8, adj_bf, preferred_element_type=jnp.float32)
        deg = colsum[0:1, :] + (1.0 - diag)
        dinv = jax.lax.rsqrt(deg)
        dcorr = dinv * (1.0 - diag)

        xgT = jnp.transpose(x_ref[g]).astype(jnp.bfloat16)
        q1 = jnp.dot(W1T, xgT,
                     preferred_element_type=jnp.float32)
        m1 = q1 * dinv
        y1 = jnp.dot(m1.astype(jnp.bfloat16), adj_bf,
                     preferred_element_type=jnp.float32)
        y1 = y1 + q1 * dcorr
        h1 = jnp.maximum(y1 * dinv + b1c, 0.0).astype(jnp.bfloat16)

        q2 = jnp.dot(W2T, h1, preferred_element_type=jnp.float32)
        m2 = q2 * dinv
        y2 = jnp.dot(m2.astype(jnp.bfloat16), adj_bf,
                     preferred_element_type=jnp.float32)
        y2 = y2 + q2 * dcorr
        h2 = jnp.maximum(y2 * dinv + b2c, 0.0)

        out_ref[pl.ds(g, 1), :] = jnp.mean(h2, axis=1)[None, :]


def kernel(adj_matrices, node_features, W1, b1, W2, b2):
    B, N, Dd = node_features.shape
    H = W1.shape[1]
    rows = N // _NSLAB

    return pl.pallas_call(
        _gnn_kernel,
        in_specs=[
            pl.BlockSpec(memory_space=pltpu.MemorySpace.HBM),
            pl.BlockSpec((B, N, Dd), lambda: (0, 0, 0)),
            pl.BlockSpec((Dd, H), lambda: (0, 0)),
            pl.BlockSpec((H,), lambda: (0,)),
            pl.BlockSpec((H, H), lambda: (0, 0)),
            pl.BlockSpec((H,), lambda: (0,)),
        ],
        out_specs=pl.BlockSpec((B, H), lambda: (0, 0)),
        out_shape=jax.ShapeDtypeStruct((B, H), jnp.float32),
        scratch_shapes=[
            pltpu.VMEM((2, _NSLAB, rows, N), jnp.float32),
            pltpu.VMEM((N, N), jnp.bfloat16),
            pltpu.SemaphoreType.DMA((2, _NSLAB)),
        ],
        compiler_params=pltpu.CompilerParams(
            vmem_limit_bytes=100 * 1024 * 1024,
        ),
    )(adj_matrices, node_features, W1, b1, W2, b2)
